# baseline (speedup 1.0000x reference)
import functools

import jax
import jax.numpy as jnp
from jax import lax
from jax.experimental import pallas as pl
from jax.experimental.pallas import tpu as pltpu
from jax.experimental.pallas import tpu_sc as plsc

D = 32
NC = 2
NS = 16
NW = NC * NS
GROUP = 128
TBS = 8192


def _linearize_table(table):
    v = table.shape[0]
    lin128 = lax.optimization_barrier(
        jnp.reshape(table, (v * D // 128, 128)))
    return jnp.reshape(lin128, (v, D))


def _sc_gather_transposed(table_lin, idx3):
    nw, cols, _ = idx3.shape
    n_fields = nw * cols // GROUP
    mesh = plsc.VectorSubcoreMesh(core_axis_name="c", subcore_axis_name="s")

    @functools.partial(
        pl.kernel,
        out_type=jax.ShapeDtypeStruct((n_fields, 4, GROUP, 8, GROUP),
                                      jnp.float32),
        mesh=mesh,
        compiler_params=pltpu.CompilerParams(use_tc_tiling_on_sc=False,
                                             needs_layout_passes=False),
        scratch_types=[
            pltpu.VMEM((cols, GROUP), jnp.int32),
            pltpu.VMEM((2 * GROUP, D), jnp.float32),
            pltpu.VMEM((2 * D, GROUP), jnp.float32),
            pltpu.SemaphoreType.DMA,
            pltpu.SemaphoreType.DMA,
        ],
    )
    def k(tab_hbm, idx_hbm, out_hbm, idx_v, rows_v, tbuf_v, sem_g, sem_w):
        wid = lax.axis_index("s") * NC + lax.axis_index("c")
        t0 = wid * cols
        pltpu.sync_copy(idx_hbm.at[wid], idx_v)
        iota16 = lax.iota(jnp.int32, 16)

        def fire_gather(c, slot):
            pltpu.async_copy(
                tab_hbm.at[idx_v.at[c]],
                rows_v.at[pl.ds(slot * GROUP, GROUP)],
                sem_g,
            )

        def wait_gather():
            pltpu.make_async_copy(
                tab_hbm.at[pl.ds(0, GROUP)],
                rows_v.at[pl.ds(0, GROUP)],
                sem_g,
            ).wait()

        def wait_write4():
            for _ in range(4):
                pltpu.make_async_copy(
                    tbuf_v.at[pl.ds(0, 8), :],
                    out_hbm.at[0, 0, 0],
                    sem_w,
                ).wait()

        fire_gather(0, 0)

        def body(c, carry):
            p = lax.rem(c, 2)
            rbase = p * GROUP
            fbase = p * D

            @pl.when(c >= 2)
            def _():
                wait_write4()

            @pl.when(c + 1 < cols)
            def _():
                fire_gather(c + 1, 1 - p)

            wait_gather()

            def trans(f, carry2):
                col_idx = jnp.full((16,), f, jnp.int32)
                for kk in range(8):
                    v = plsc.load_gather(
                        rows_v, [iota16 + (kk * 16 + rbase), col_idx])
                    tbuf_v[fbase + f, pl.ds(kk * 16, 16)] = v
                return carry2

            lax.fori_loop(0, D, trans, 0)

            t = t0 + c
            i = lax.div(t, GROUP)
            bb = lax.rem(t, GROUP)
            for fg in range(4):
                pltpu.async_copy(
                    tbuf_v.at[pl.ds(fbase + fg * 8, 8), :],
                    out_hbm.at[i, fg, bb],
                    sem_w,
                )
            return carry

        lax.fori_loop(0, cols, body, 0)
        wait_write4()
        wait_write4()

    return k(table_lin, idx3)


def kernel(input, embedding_matrix):
    batch, n_fields = input.shape
    vocab = embedding_matrix.shape[0]
    cols = batch * n_fields // (NW * GROUP)

    table_lin = _linearize_table(embedding_matrix)
    idx3 = jnp.reshape(jnp.transpose(input).astype(jnp.int32),
                       (NW, cols, GROUP))
    out5 = _sc_gather_transposed(table_lin, idx3)
    return out5.transpose(2, 4, 0, 1, 3).reshape(batch, n_fields, D)


# speedup vs baseline: 1.0811x; 1.0811x over previous
import functools

import jax
import jax.numpy as jnp
from jax import lax
from jax.experimental import pallas as pl
from jax.experimental.pallas import tpu as pltpu
from jax.experimental.pallas import tpu_sc as plsc

D = 32
NC = 2
NS = 16
NW = NC * NS
GROUP = 128
TBS = 8192


def _linearize_table(table):
    v = table.shape[0]
    lin128 = lax.optimization_barrier(
        jnp.reshape(table, (v * D // 128, 128)))
    return jnp.reshape(lin128, (v, D))


def _sc_gather_transposed(table_lin, idx3):
    nw, cols, _ = idx3.shape
    n_fields = nw * cols // GROUP
    mesh = plsc.VectorSubcoreMesh(core_axis_name="c", subcore_axis_name="s")

    nslot = 4
    npair = cols // nslot

    @functools.partial(
        pl.kernel,
        out_type=jax.ShapeDtypeStruct((n_fields, 4, GROUP, 8, GROUP),
                                      jnp.float32),
        mesh=mesh,
        compiler_params=pltpu.CompilerParams(use_tc_tiling_on_sc=False,
                                             needs_layout_passes=False),
        scratch_types=[
            pltpu.VMEM((cols, GROUP), jnp.int32),
            pltpu.VMEM((nslot * GROUP, D), jnp.float32),
            pltpu.VMEM((nslot * D, GROUP), jnp.float32),
            pltpu.SemaphoreType.DMA,
            pltpu.SemaphoreType.DMA,
        ],
    )
    def k(tab_hbm, idx_hbm, out_hbm, idx_v, rows_v, tbuf_v, sem_g, sem_w):
        wid = lax.axis_index("s") * NC + lax.axis_index("c")
        t0 = wid * cols
        pltpu.sync_copy(idx_hbm.at[wid], idx_v)
        iota16 = lax.iota(jnp.int32, 16)
        ridx = [[iota16 + (s * GROUP + kk * 16) for kk in range(8)]
                for s in range(nslot)]

        def fire_gather(c, slot):
            pltpu.async_copy(
                tab_hbm.at[idx_v.at[c]],
                rows_v.at[pl.ds(slot * GROUP, GROUP)],
                sem_g,
            )

        def wait_gather():
            pltpu.make_async_copy(
                tab_hbm.at[pl.ds(0, GROUP)],
                rows_v.at[pl.ds(0, GROUP)],
                sem_g,
            ).wait()

        def wait_write4():
            for _ in range(4):
                pltpu.make_async_copy(
                    tbuf_v.at[pl.ds(0, 8), :],
                    out_hbm.at[0, 0, 0],
                    sem_w,
                ).wait()

        for s in range(nslot - 1):
            fire_gather(s, s)

        def body(cc, carry):
            for s in range(nslot):
                c = cc * nslot + s

                @pl.when(c >= nslot)
                def _():
                    wait_write4()

                @pl.when(c + nslot - 1 < cols)
                def _():
                    fire_gather(c + nslot - 1, (s + nslot - 1) % nslot)

                wait_gather()

                for f in range(D):
                    cidx = jnp.full((16,), f, jnp.int32)
                    vs = [plsc.load_gather(rows_v, [ridx[s][kk], cidx])
                          for kk in range(8)]
                    for kk in range(8):
                        tbuf_v[s * D + f, pl.ds(kk * 16, 16)] = vs[kk]

                t = t0 + c
                i = lax.div(t, GROUP)
                bb = lax.rem(t, GROUP)
                for fg in range(4):
                    pltpu.async_copy(
                        tbuf_v.at[pl.ds(s * D + fg * 8, 8), :],
                        out_hbm.at[i, fg, bb],
                        sem_w,
                    )
            return carry

        lax.fori_loop(0, npair, body, 0)
        for _ in range(nslot):
            wait_write4()

    return k(table_lin, idx3)


def kernel(input, embedding_matrix):
    batch, n_fields = input.shape
    vocab = embedding_matrix.shape[0]
    cols = batch * n_fields // (NW * GROUP)

    table_lin = _linearize_table(embedding_matrix)
    idx3 = jnp.reshape(jnp.transpose(input).astype(jnp.int32),
                       (NW, cols, GROUP))
    out5 = _sc_gather_transposed(table_lin, idx3)
    return out5.transpose(2, 4, 0, 1, 3).reshape(batch, n_fields, D)
